# baseline (speedup 1.0000x reference)
import functools

import jax
import jax.numpy as jnp
from jax import lax
from jax.experimental import pallas as pl
from jax.experimental.pallas import tpu as pltpu
from jax.experimental.pallas import tpu_sc as plsc

N_NODES = 10000
D = 128
N_PAD = 10240
ROWS_PER_TILE = N_PAD // 16
N_EDGES = 320000
CHUNK = 128
NW = 32
CHUNKS_PER_W = 80
E_PAD = NW * CHUNKS_PER_W * CHUNK
DUMMY = N_NODES
DEG_W = 128

_mesh = plsc.VectorSubcoreMesh(core_axis_name="c", subcore_axis_name="s")


NBUF = 2
SBLK = 40
SBLOCKS = ((0, 40), (40, 40))


def _deg_body(dst_hbm, ones_hbm, zeros_hbm, out_hbm, dstv, onesv, sem, S):
    c = lax.axis_index("c")
    s = lax.axis_index("s")
    w = c * 16 + s
    r0 = s * ROWS_PER_TILE
    pltpu.sync_copy(zeros_hbm, S.at[pl.ds(r0, ROWS_PER_TILE)])
    pltpu.sync_copy(ones_hbm, onesv)
    pltpu.sync_copy(dst_hbm.at[w], dstv)
    plsc.subcore_barrier()

    def fire(i, carry):
        pltpu.async_copy(onesv, S.at[dstv.at[i]], sem, add=True)
        return carry

    lax.fori_loop(0, CHUNKS_PER_W, fire, 0)

    def drain(i, carry):
        pltpu.make_async_copy(onesv, S.at[dstv.at[i]], sem).wait()
        return carry

    lax.fori_loop(0, CHUNKS_PER_W, drain, 0)
    plsc.subcore_barrier()
    pltpu.sync_copy(S.at[pl.ds(r0, ROWS_PER_TILE)],
                    out_hbm.at[pl.ds(c * N_PAD + r0, ROWS_PER_TILE)])


_deg_call = functools.partial(
    pl.kernel,
    mesh=_mesh,
    out_type=jax.ShapeDtypeStruct((2 * N_PAD, DEG_W), jnp.float32),
    scratch_types=[
        pltpu.VMEM((CHUNKS_PER_W, CHUNK), jnp.int32),
        pltpu.VMEM((CHUNK, DEG_W), jnp.float32),
        pltpu.SemaphoreType.DMA,
        pltpu.VMEM_SHARED((N_PAD, DEG_W), jnp.float32),
    ],
)(_deg_body)


def _prop_body(g_hbm, src_hbm, dst_hbm, zeros_hbm, out_hbm,
               srcv, dstv, rows0, rows1, gsem, ssem, S):
    c = lax.axis_index("c")
    s = lax.axis_index("s")
    w = c * 16 + s
    r0 = s * ROWS_PER_TILE
    rows = (rows0, rows1)
    pltpu.sync_copy(zeros_hbm, S.at[pl.ds(r0, ROWS_PER_TILE)])
    pltpu.sync_copy(dst_hbm.at[w], dstv)
    plsc.subcore_barrier()

    for off, n in SBLOCKS:
        pltpu.sync_copy(src_hbm.at[w, pl.ds(off, n)],
                        srcv.at[pl.ds(0, n)])
        pltpu.async_copy(g_hbm.at[srcv.at[0]], rows[0], gsem)

        def body(t, carry):
            for b in range(NBUF):
                lc = t * NBUF + b
                gc = off + lc
                pltpu.make_async_copy(g_hbm.at[srcv.at[lc]], rows[b],
                                      gsem).wait()
                pltpu.async_copy(rows[b], S.at[dstv.at[gc]], ssem,
                                 add=True)

                @pl.when(lc >= 1)
                def _():
                    pltpu.make_async_copy(rows[1 - b],
                                          S.at[dstv.at[gc - 1]],
                                          ssem).wait()

                @pl.when(lc + 1 < n)
                def _():
                    pltpu.async_copy(g_hbm.at[srcv.at[lc + 1]],
                                     rows[1 - b], gsem)
            return carry

        lax.fori_loop(0, n // NBUF, body, 0)
        pltpu.make_async_copy(rows[(n - 1) % NBUF],
                              S.at[dstv.at[off + n - 1]], ssem).wait()

    plsc.subcore_barrier()
    pltpu.sync_copy(S.at[pl.ds(r0, ROWS_PER_TILE)],
                    out_hbm.at[pl.ds(c * N_PAD + r0, ROWS_PER_TILE)])


_prop_call = functools.partial(
    pl.kernel,
    mesh=_mesh,
    out_type=jax.ShapeDtypeStruct((2 * N_PAD, D), jnp.float32),
    scratch_types=[
        pltpu.VMEM((SBLK, CHUNK), jnp.int32),
        pltpu.VMEM((CHUNKS_PER_W, CHUNK), jnp.int32),
        pltpu.VMEM((CHUNK, D), jnp.float32),
        pltpu.VMEM((CHUNK, D), jnp.float32),
        pltpu.SemaphoreType.DMA,
        pltpu.SemaphoreType.DMA,
        pltpu.VMEM_SHARED((N_PAD, D), jnp.float32),
    ],
)(_prop_body)


BLK = 640


def _dinv_from(degp_ref):
    deg = degp_ref[0, :, 0:1] + degp_ref[1, :, 0:1] + 1.0
    return lax.rsqrt(deg)


def _tc_a_body(x_ref, w_ref, degp_ref, g_ref):
    dinv = _dinv_from(degp_ref)
    g_ref[...] = dinv * jnp.dot(x_ref[...], w_ref[...],
                                preferred_element_type=jnp.float32)


_tc_a = pl.pallas_call(
    _tc_a_body,
    grid=(N_PAD // BLK,),
    in_specs=[
        pl.BlockSpec((BLK, D), lambda i: (i, 0)),
        pl.BlockSpec((D, D), lambda i: (0, 0)),
        pl.BlockSpec((2, BLK, DEG_W), lambda i: (0, i, 0)),
    ],
    out_specs=pl.BlockSpec((BLK, D), lambda i: (i, 0)),
    out_shape=jax.ShapeDtypeStruct((N_PAD, D), jnp.float32),
)


def _tc_b_body(sp_ref, g1_ref, degp_ref, b_ref, w_ref, g_ref):
    dinv = _dinv_from(degp_ref)
    agg = sp_ref[0] + sp_ref[1] + g1_ref[...]
    z = jnp.maximum(dinv * agg + b_ref[...], 0.0)
    g_ref[...] = dinv * jnp.dot(z, w_ref[...],
                                preferred_element_type=jnp.float32)


_tc_b = pl.pallas_call(
    _tc_b_body,
    grid=(N_PAD // BLK,),
    in_specs=[
        pl.BlockSpec((2, BLK, D), lambda i: (0, i, 0)),
        pl.BlockSpec((BLK, D), lambda i: (i, 0)),
        pl.BlockSpec((2, BLK, DEG_W), lambda i: (0, i, 0)),
        pl.BlockSpec((1, D), lambda i: (0, 0)),
        pl.BlockSpec((D, D), lambda i: (0, 0)),
    ],
    out_specs=pl.BlockSpec((BLK, D), lambda i: (i, 0)),
    out_shape=jax.ShapeDtypeStruct((N_PAD, D), jnp.float32),
)


def _tc_c_body(sp_ref, g2_ref, degp_ref, b_ref, o_ref):
    dinv = _dinv_from(degp_ref)
    agg = sp_ref[0] + sp_ref[1] + g2_ref[...]
    o_ref[...] = dinv * agg + b_ref[...]


_tc_c = pl.pallas_call(
    _tc_c_body,
    grid=(N_PAD // BLK,),
    in_specs=[
        pl.BlockSpec((2, BLK, D), lambda i: (0, i, 0)),
        pl.BlockSpec((BLK, D), lambda i: (i, 0)),
        pl.BlockSpec((2, BLK, DEG_W), lambda i: (0, i, 0)),
        pl.BlockSpec((1, D), lambda i: (0, 0)),
    ],
    out_specs=pl.BlockSpec((BLK, D), lambda i: (i, 0)),
    out_shape=jax.ShapeDtypeStruct((N_PAD, D), jnp.float32),
)


def kernel(x, edge_index, W1, b1, W2, b2):
    ei = edge_index.astype(jnp.int32)
    padv = DUMMY + jnp.arange(E_PAD - N_EDGES, dtype=jnp.int32) % (
        N_PAD - N_NODES)
    src = jnp.concatenate([ei[0], padv]).reshape(NW, CHUNKS_PER_W, CHUNK)
    dst = jnp.concatenate([ei[1], padv]).reshape(NW, CHUNKS_PER_W, CHUNK)

    x_pad = jnp.pad(x, ((0, N_PAD - N_NODES), (0, 0)))
    ones_deg = jnp.ones((CHUNK, DEG_W), jnp.float32)
    zeros_row = jnp.zeros((ROWS_PER_TILE, D), jnp.float32)

    degp = _deg_call(dst, ones_deg, zeros_row).reshape(2, N_PAD, DEG_W)
    g1 = _tc_a(x_pad, W1, degp)
    s1 = _prop_call(g1, src, dst, zeros_row).reshape(2, N_PAD, D)
    g2 = _tc_b(s1, g1, degp, b1.reshape(1, D), W2)
    s2 = _prop_call(g2, src, dst, zeros_row).reshape(2, N_PAD, D)
    out = _tc_c(s2, g2, degp, b2.reshape(1, D))
    return out[:N_NODES]


# speedup vs baseline: 1.1164x; 1.1164x over previous
import functools

import jax
import jax.numpy as jnp
from jax import lax
from jax.experimental import pallas as pl
from jax.experimental.pallas import tpu as pltpu
from jax.experimental.pallas import tpu_sc as plsc

N_NODES = 10000
D = 128
N_PAD = 10240
ROWS_PER_TILE = N_PAD // 16
N_EDGES = 320000
CHUNK = 128
NW = 32
CHUNKS_PER_W = 80
E_PAD = NW * CHUNKS_PER_W * CHUNK
DUMMY = N_NODES
DEG_W = 128

_mesh = plsc.VectorSubcoreMesh(core_axis_name="c", subcore_axis_name="s")


NBUF = 2
SBLK = 40
SBLOCKS = ((0, 40), (40, 40))


def _deg_body(dst_hbm, ones_hbm, zeros_hbm, out_hbm, dstv, onesv, sem, S):
    c = lax.axis_index("c")
    s = lax.axis_index("s")
    w = c * 16 + s
    r0 = s * ROWS_PER_TILE
    pltpu.sync_copy(zeros_hbm, S.at[pl.ds(r0, ROWS_PER_TILE)])
    pltpu.sync_copy(ones_hbm, onesv)
    pltpu.sync_copy(dst_hbm.at[w], dstv)
    plsc.subcore_barrier()

    def fire(i, carry):
        pltpu.async_copy(onesv, S.at[dstv.at[i]], sem, add=True)
        return carry

    lax.fori_loop(0, CHUNKS_PER_W, fire, 0)

    def drain(i, carry):
        pltpu.make_async_copy(onesv, S.at[dstv.at[i]], sem).wait()
        return carry

    lax.fori_loop(0, CHUNKS_PER_W, drain, 0)
    plsc.subcore_barrier()
    pltpu.sync_copy(S.at[pl.ds(r0, ROWS_PER_TILE)],
                    out_hbm.at[pl.ds(c * N_PAD + r0, ROWS_PER_TILE)])


_deg_call = functools.partial(
    pl.kernel,
    mesh=_mesh,
    out_type=jax.ShapeDtypeStruct((2 * N_PAD, DEG_W), jnp.float32),
    scratch_types=[
        pltpu.VMEM((CHUNKS_PER_W, CHUNK), jnp.int32),
        pltpu.VMEM((CHUNK, DEG_W), jnp.float32),
        pltpu.SemaphoreType.DMA,
        pltpu.VMEM_SHARED((N_PAD, DEG_W), jnp.float32),
    ],
)(_deg_body)


def _prop_body(g_hbm, src_hbm, dst_hbm, zeros_hbm, out_hbm,
               srcv, dstv, rows0, rows1, gsem, ssem, S):
    c = lax.axis_index("c")
    s = lax.axis_index("s")
    w = c * 16 + s
    r0 = s * ROWS_PER_TILE
    rows = (rows0, rows1)
    pltpu.sync_copy(zeros_hbm, S.at[pl.ds(r0, ROWS_PER_TILE)])
    pltpu.sync_copy(dst_hbm.at[w], dstv)
    plsc.subcore_barrier()

    for off, n in SBLOCKS:
        pltpu.sync_copy(src_hbm.at[w, pl.ds(off, n)],
                        srcv.at[pl.ds(0, n)])
        for b in range(NBUF):
            pltpu.async_copy(g_hbm.at[srcv.at[b]], rows[b],
                             (gsem, ssem)[b])

        def body(t, carry):
            for b in range(NBUF):
                lc = t * NBUF + b
                gc = off + lc
                sem = (gsem, ssem)[b]
                pltpu.make_async_copy(g_hbm.at[srcv.at[lc]], rows[b],
                                      sem).wait()
                pltpu.sync_copy(rows[b], S.at[dstv.at[gc]], add=True)
                nxt = lc + NBUF

                @pl.when(nxt < n)
                def _():
                    pltpu.async_copy(g_hbm.at[srcv.at[nxt]], rows[b],
                                     sem)
            return carry

        lax.fori_loop(0, n // NBUF, body, 0)

    plsc.subcore_barrier()
    pltpu.sync_copy(S.at[pl.ds(r0, ROWS_PER_TILE)],
                    out_hbm.at[pl.ds(c * N_PAD + r0, ROWS_PER_TILE)])


_prop_call = functools.partial(
    pl.kernel,
    mesh=_mesh,
    out_type=jax.ShapeDtypeStruct((2 * N_PAD, D), jnp.float32),
    scratch_types=[
        pltpu.VMEM((SBLK, CHUNK), jnp.int32),
        pltpu.VMEM((CHUNKS_PER_W, CHUNK), jnp.int32),
        pltpu.VMEM((CHUNK, D), jnp.float32),
        pltpu.VMEM((CHUNK, D), jnp.float32),
        pltpu.SemaphoreType.DMA,
        pltpu.SemaphoreType.DMA,
        pltpu.VMEM_SHARED((N_PAD, D), jnp.float32),
    ],
)(_prop_body)


BLK = 640


def _dinv_from(degp_ref):
    deg = degp_ref[0, :, 0:1] + degp_ref[1, :, 0:1] + 1.0
    return lax.rsqrt(deg)


def _tc_a_body(x_ref, w_ref, degp_ref, g_ref):
    dinv = _dinv_from(degp_ref)
    g_ref[...] = dinv * jnp.dot(x_ref[...], w_ref[...],
                                preferred_element_type=jnp.float32)


_tc_a = pl.pallas_call(
    _tc_a_body,
    grid=(N_PAD // BLK,),
    in_specs=[
        pl.BlockSpec((BLK, D), lambda i: (i, 0)),
        pl.BlockSpec((D, D), lambda i: (0, 0)),
        pl.BlockSpec((2, BLK, DEG_W), lambda i: (0, i, 0)),
    ],
    out_specs=pl.BlockSpec((BLK, D), lambda i: (i, 0)),
    out_shape=jax.ShapeDtypeStruct((N_PAD, D), jnp.float32),
)


def _tc_b_body(sp_ref, g1_ref, degp_ref, b_ref, w_ref, g_ref):
    dinv = _dinv_from(degp_ref)
    agg = sp_ref[0] + sp_ref[1] + g1_ref[...]
    z = jnp.maximum(dinv * agg + b_ref[...], 0.0)
    g_ref[...] = dinv * jnp.dot(z, w_ref[...],
                                preferred_element_type=jnp.float32)


_tc_b = pl.pallas_call(
    _tc_b_body,
    grid=(N_PAD // BLK,),
    in_specs=[
        pl.BlockSpec((2, BLK, D), lambda i: (0, i, 0)),
        pl.BlockSpec((BLK, D), lambda i: (i, 0)),
        pl.BlockSpec((2, BLK, DEG_W), lambda i: (0, i, 0)),
        pl.BlockSpec((1, D), lambda i: (0, 0)),
        pl.BlockSpec((D, D), lambda i: (0, 0)),
    ],
    out_specs=pl.BlockSpec((BLK, D), lambda i: (i, 0)),
    out_shape=jax.ShapeDtypeStruct((N_PAD, D), jnp.float32),
)


def _tc_c_body(sp_ref, g2_ref, degp_ref, b_ref, o_ref):
    dinv = _dinv_from(degp_ref)
    agg = sp_ref[0] + sp_ref[1] + g2_ref[...]
    o_ref[...] = dinv * agg + b_ref[...]


_tc_c = pl.pallas_call(
    _tc_c_body,
    grid=(N_PAD // BLK,),
    in_specs=[
        pl.BlockSpec((2, BLK, D), lambda i: (0, i, 0)),
        pl.BlockSpec((BLK, D), lambda i: (i, 0)),
        pl.BlockSpec((2, BLK, DEG_W), lambda i: (0, i, 0)),
        pl.BlockSpec((1, D), lambda i: (0, 0)),
    ],
    out_specs=pl.BlockSpec((BLK, D), lambda i: (i, 0)),
    out_shape=jax.ShapeDtypeStruct((N_PAD, D), jnp.float32),
)


def kernel(x, edge_index, W1, b1, W2, b2):
    ei = edge_index.astype(jnp.int32)
    padv = DUMMY + jnp.arange(E_PAD - N_EDGES, dtype=jnp.int32) % (
        N_PAD - N_NODES)
    src = jnp.concatenate([ei[0], padv]).reshape(NW, CHUNKS_PER_W, CHUNK)
    dst = jnp.concatenate([ei[1], padv]).reshape(NW, CHUNKS_PER_W, CHUNK)

    x_pad = jnp.pad(x, ((0, N_PAD - N_NODES), (0, 0)))
    ones_deg = jnp.ones((CHUNK, DEG_W), jnp.float32)
    zeros_row = jnp.zeros((ROWS_PER_TILE, D), jnp.float32)

    degp = _deg_call(dst, ones_deg, zeros_row).reshape(2, N_PAD, DEG_W)
    g1 = _tc_a(x_pad, W1, degp)
    s1 = _prop_call(g1, src, dst, zeros_row).reshape(2, N_PAD, D)
    g2 = _tc_b(s1, g1, degp, b1.reshape(1, D), W2)
    s2 = _prop_call(g2, src, dst, zeros_row).reshape(2, N_PAD, D)
    out = _tc_c(s2, g2, degp, b2.reshape(1, D))
    return out[:N_NODES]


# speedup vs baseline: 1.1587x; 1.0379x over previous
import functools

import numpy as np
import jax
import jax.numpy as jnp
from jax import lax
from jax.experimental import pallas as pl
from jax.experimental.pallas import tpu as pltpu
from jax.experimental.pallas import tpu_sc as plsc

N_NODES = 10000
D = 128
N_PAD = 10240
ROWS_PER_TILE = N_PAD // 16
N_EDGES = 320000
CHUNK = 128
NW = 32
CHUNKS_PER_W = 80
E_PAD = NW * CHUNKS_PER_W * CHUNK
DUMMY = N_NODES
DEG_W = 128

_mesh = plsc.VectorSubcoreMesh(core_axis_name="c", subcore_axis_name="s")


NBUF = 2
SBLK = 40
SBLOCKS = ((0, 40), (40, 40))


def _deg_body(dst_hbm, ones_hbm, zeros_hbm, out_hbm, dstv, onesv, sem, S):
    c = lax.axis_index("c")
    s = lax.axis_index("s")
    w = c * 16 + s
    r0 = s * ROWS_PER_TILE
    pltpu.sync_copy(zeros_hbm, S.at[pl.ds(r0, ROWS_PER_TILE)])
    pltpu.sync_copy(ones_hbm, onesv)
    pltpu.sync_copy(dst_hbm.at[w], dstv)
    plsc.subcore_barrier()

    def fire(i, carry):
        pltpu.async_copy(onesv, S.at[dstv.at[i]], sem, add=True)
        return carry

    lax.fori_loop(0, CHUNKS_PER_W, fire, 0)

    def drain(i, carry):
        pltpu.make_async_copy(onesv, S.at[dstv.at[i]], sem).wait()
        return carry

    lax.fori_loop(0, CHUNKS_PER_W, drain, 0)
    plsc.subcore_barrier()
    pltpu.sync_copy(S.at[pl.ds(r0, ROWS_PER_TILE)],
                    out_hbm.at[pl.ds(c * N_PAD + r0, ROWS_PER_TILE)])


_deg_call = functools.partial(
    pl.kernel,
    mesh=_mesh,
    out_type=jax.ShapeDtypeStruct((2 * N_PAD, DEG_W), jnp.float32),
    scratch_types=[
        pltpu.VMEM((CHUNKS_PER_W, CHUNK), jnp.int32),
        pltpu.VMEM((CHUNK, DEG_W), jnp.float32),
        pltpu.SemaphoreType.DMA,
        pltpu.VMEM_SHARED((N_PAD, DEG_W), jnp.float32),
    ],
)(_deg_body)


def _prop_body(g_hbm, src_hbm, dst_hbm, zeros_hbm, out_hbm,
               srcv, dstv, rows0, rows1, gsem, ssem, S):
    c = lax.axis_index("c")
    s = lax.axis_index("s")
    w = c * 16 + s
    r0 = s * ROWS_PER_TILE
    rows = (rows0, rows1)
    pltpu.sync_copy(zeros_hbm, S.at[pl.ds(r0, ROWS_PER_TILE)])
    pltpu.sync_copy(dst_hbm.at[w], dstv)
    plsc.subcore_barrier()

    for off, n in SBLOCKS:
        pltpu.sync_copy(src_hbm.at[w, pl.ds(off, n)],
                        srcv.at[pl.ds(0, n)])
        for b in range(NBUF):
            pltpu.async_copy(g_hbm.at[srcv.at[b]], rows[b],
                             (gsem, ssem)[b])

        def body(t, carry):
            for b in range(NBUF):
                lc = t * NBUF + b
                gc = off + lc
                sem = (gsem, ssem)[b]
                pltpu.make_async_copy(g_hbm.at[srcv.at[lc]], rows[b],
                                      sem).wait()
                pltpu.sync_copy(rows[b], S.at[dstv.at[gc]], add=True)
                nxt = lc + NBUF

                @pl.when(nxt < n)
                def _():
                    pltpu.async_copy(g_hbm.at[srcv.at[nxt]], rows[b],
                                     sem)
            return carry

        lax.fori_loop(0, n // NBUF, body, 0)

    plsc.subcore_barrier()
    pltpu.sync_copy(S.at[pl.ds(r0, ROWS_PER_TILE)],
                    out_hbm.at[pl.ds(c * N_PAD + r0, ROWS_PER_TILE)])


_prop_call = functools.partial(
    pl.kernel,
    mesh=_mesh,
    out_type=jax.ShapeDtypeStruct((2 * N_PAD, D), jnp.float32),
    scratch_types=[
        pltpu.VMEM((SBLK, CHUNK), jnp.int32),
        pltpu.VMEM((CHUNKS_PER_W, CHUNK), jnp.int32),
        pltpu.VMEM((CHUNK, D), jnp.float32),
        pltpu.VMEM((CHUNK, D), jnp.float32),
        pltpu.SemaphoreType.DMA,
        pltpu.SemaphoreType.DMA,
        pltpu.VMEM_SHARED((N_PAD, D), jnp.float32),
    ],
)(_prop_body)


BLK = 1280


def _dinv_from(degp_ref):
    deg = degp_ref[0, :, 0:1] + degp_ref[1, :, 0:1] + 1.0
    return lax.rsqrt(deg)


def _tc_a_body(x_ref, w_ref, degp_ref, g_ref):
    dinv = _dinv_from(degp_ref)
    g_ref[...] = dinv * jnp.dot(x_ref[...], w_ref[...],
                                preferred_element_type=jnp.float32)


_tc_a = pl.pallas_call(
    _tc_a_body,
    grid=(N_PAD // BLK,),
    in_specs=[
        pl.BlockSpec((BLK, D), lambda i: (i, 0)),
        pl.BlockSpec((D, D), lambda i: (0, 0)),
        pl.BlockSpec((2, BLK, DEG_W), lambda i: (0, i, 0)),
    ],
    out_specs=pl.BlockSpec((BLK, D), lambda i: (i, 0)),
    out_shape=jax.ShapeDtypeStruct((N_PAD, D), jnp.float32),
)


def _tc_b_body(sp_ref, g1_ref, degp_ref, b_ref, w_ref, g_ref):
    dinv = _dinv_from(degp_ref)
    agg = sp_ref[0] + sp_ref[1] + g1_ref[...]
    z = jnp.maximum(dinv * agg + b_ref[...], 0.0)
    g_ref[...] = dinv * jnp.dot(z, w_ref[...],
                                preferred_element_type=jnp.float32)


_tc_b = pl.pallas_call(
    _tc_b_body,
    grid=(N_PAD // BLK,),
    in_specs=[
        pl.BlockSpec((2, BLK, D), lambda i: (0, i, 0)),
        pl.BlockSpec((BLK, D), lambda i: (i, 0)),
        pl.BlockSpec((2, BLK, DEG_W), lambda i: (0, i, 0)),
        pl.BlockSpec((1, D), lambda i: (0, 0)),
        pl.BlockSpec((D, D), lambda i: (0, 0)),
    ],
    out_specs=pl.BlockSpec((BLK, D), lambda i: (i, 0)),
    out_shape=jax.ShapeDtypeStruct((N_PAD, D), jnp.float32),
)


def _tc_c_body(sp_ref, g2_ref, degp_ref, b_ref, o_ref):
    dinv = _dinv_from(degp_ref)
    agg = sp_ref[0] + sp_ref[1] + g2_ref[...]
    o_ref[...] = dinv * agg + b_ref[...]


_tc_c = pl.pallas_call(
    _tc_c_body,
    grid=(N_PAD // BLK,),
    in_specs=[
        pl.BlockSpec((2, BLK, D), lambda i: (0, i, 0)),
        pl.BlockSpec((BLK, D), lambda i: (i, 0)),
        pl.BlockSpec((2, BLK, DEG_W), lambda i: (0, i, 0)),
        pl.BlockSpec((1, D), lambda i: (0, 0)),
    ],
    out_specs=pl.BlockSpec((BLK, D), lambda i: (i, 0)),
    out_shape=jax.ShapeDtypeStruct((N_PAD, D), jnp.float32),
)


def kernel(x, edge_index, W1, b1, W2, b2):
    ei = edge_index.astype(jnp.int32)
    padv = jnp.asarray(
        DUMMY + np.arange(E_PAD - N_EDGES, dtype=np.int32) % (
            N_PAD - N_NODES))
    src = jnp.concatenate([ei[0], padv]).reshape(NW, CHUNKS_PER_W, CHUNK)
    dst = jnp.concatenate([ei[1], padv]).reshape(NW, CHUNKS_PER_W, CHUNK)

    x_pad = jnp.pad(x, ((0, N_PAD - N_NODES), (0, 0)))
    ones_deg = jnp.ones((CHUNK, DEG_W), jnp.float32)
    zeros_row = jnp.zeros((ROWS_PER_TILE, D), jnp.float32)

    degp = _deg_call(dst, ones_deg, zeros_row).reshape(2, N_PAD, DEG_W)
    g1 = _tc_a(x_pad, W1, degp)
    s1 = _prop_call(g1, src, dst, zeros_row).reshape(2, N_PAD, D)
    g2 = _tc_b(s1, g1, degp, b1.reshape(1, D), W2)
    s2 = _prop_call(g2, src, dst, zeros_row).reshape(2, N_PAD, D)
    out = _tc_c(s2, g2, degp, b2.reshape(1, D))
    return out[:N_NODES]


# speedup vs baseline: 1.1594x; 1.0006x over previous
import functools

import numpy as np
import jax
import jax.numpy as jnp
from jax import lax
from jax.experimental import pallas as pl
from jax.experimental.pallas import tpu as pltpu
from jax.experimental.pallas import tpu_sc as plsc

N_NODES = 10000
D = 128
N_PAD = 10240
ROWS_PER_TILE = N_PAD // 16
N_EDGES = 320000
CHUNK = 128
NW = 32
CHUNKS_PER_W = 80
E_PAD = NW * CHUNKS_PER_W * CHUNK
DUMMY = N_NODES
DEG_W = 128

_mesh = plsc.VectorSubcoreMesh(core_axis_name="c", subcore_axis_name="s")


NBUF = 2
SBLK = 40
SBLOCKS = ((0, 40), (40, 40))


def _deg_body(dst_hbm, ones_hbm, zeros_hbm, out_hbm, dstv, onesv, sem, S):
    c = lax.axis_index("c")
    s = lax.axis_index("s")
    w = c * 16 + s
    r0 = s * ROWS_PER_TILE
    pltpu.sync_copy(zeros_hbm, S.at[pl.ds(r0, ROWS_PER_TILE)])
    pltpu.sync_copy(ones_hbm, onesv)
    pltpu.sync_copy(dst_hbm.at[w], dstv)
    plsc.subcore_barrier()

    def fire(i, carry):
        pltpu.async_copy(onesv, S.at[dstv.at[i]], sem, add=True)
        return carry

    lax.fori_loop(0, CHUNKS_PER_W, fire, 0)

    def drain(i, carry):
        pltpu.make_async_copy(onesv, S.at[dstv.at[i]], sem).wait()
        return carry

    lax.fori_loop(0, CHUNKS_PER_W, drain, 0)
    plsc.subcore_barrier()
    pltpu.sync_copy(S.at[pl.ds(r0, ROWS_PER_TILE)],
                    out_hbm.at[pl.ds(c * N_PAD + r0, ROWS_PER_TILE)])


_deg_call = functools.partial(
    pl.kernel,
    mesh=_mesh,
    out_type=jax.ShapeDtypeStruct((2 * N_PAD, DEG_W), jnp.float32),
    scratch_types=[
        pltpu.VMEM((CHUNKS_PER_W, CHUNK), jnp.int32),
        pltpu.VMEM((CHUNK, DEG_W), jnp.float32),
        pltpu.SemaphoreType.DMA,
        pltpu.VMEM_SHARED((N_PAD, DEG_W), jnp.float32),
    ],
)(_deg_body)


def _prop_body(g_hbm, src_hbm, dst_hbm, zeros_hbm, out_hbm,
               srcv, dstv, rows0, rows1, gsem, ssem, S):
    c = lax.axis_index("c")
    s = lax.axis_index("s")
    w = c * 16 + s
    r0 = s * ROWS_PER_TILE
    rows = (rows0, rows1)
    pltpu.sync_copy(zeros_hbm, S.at[pl.ds(r0, ROWS_PER_TILE)])
    pltpu.sync_copy(dst_hbm.at[w], dstv)
    plsc.subcore_barrier()

    for off, n in SBLOCKS:
        pltpu.sync_copy(src_hbm.at[w, pl.ds(off, n)],
                        srcv.at[pl.ds(0, n)])
        for b in range(NBUF):
            pltpu.async_copy(g_hbm.at[srcv.at[b]], rows[b],
                             (gsem, ssem)[b])

        def body(t, carry):
            for b in range(NBUF):
                lc = t * NBUF + b
                gc = off + lc
                sem = (gsem, ssem)[b]
                pltpu.make_async_copy(g_hbm.at[srcv.at[lc]], rows[b],
                                      sem).wait()
                pltpu.sync_copy(rows[b], S.at[dstv.at[gc]], add=True)
                nxt = lc + NBUF

                @pl.when(nxt < n)
                def _():
                    pltpu.async_copy(g_hbm.at[srcv.at[nxt]], rows[b],
                                     sem)
            return carry

        lax.fori_loop(0, n // NBUF, body, 0)

    plsc.subcore_barrier()
    pltpu.sync_copy(S.at[pl.ds(r0, ROWS_PER_TILE)],
                    out_hbm.at[pl.ds(c * N_PAD + r0, ROWS_PER_TILE)])


_prop_call = functools.partial(
    pl.kernel,
    mesh=_mesh,
    out_type=jax.ShapeDtypeStruct((2 * N_PAD, D), jnp.float32),
    scratch_types=[
        pltpu.VMEM((SBLK, CHUNK), jnp.int32),
        pltpu.VMEM((CHUNKS_PER_W, CHUNK), jnp.int32),
        pltpu.VMEM((CHUNK, D), jnp.float32),
        pltpu.VMEM((CHUNK, D), jnp.float32),
        pltpu.SemaphoreType.DMA,
        pltpu.SemaphoreType.DMA,
        pltpu.VMEM_SHARED((N_PAD, D), jnp.float32),
    ],
)(_prop_body)


BLK = 1280


def _dinv_from(degp_ref):
    deg = degp_ref[0, :, 0:1] + degp_ref[1, :, 0:1] + 1.0
    return lax.rsqrt(deg)


def _tc_a_body(x_ref, w_ref, degp_ref, g_ref):
    dinv = _dinv_from(degp_ref)
    g_ref[...] = dinv * jnp.dot(x_ref[...], w_ref[...],
                                preferred_element_type=jnp.float32)


_tc_a = pl.pallas_call(
    _tc_a_body,
    grid=(N_PAD // BLK,),
    in_specs=[
        pl.BlockSpec((BLK, D), lambda i: (i, 0)),
        pl.BlockSpec((D, D), lambda i: (0, 0)),
        pl.BlockSpec((2, BLK, DEG_W), lambda i: (0, i, 0)),
    ],
    out_specs=pl.BlockSpec((BLK, D), lambda i: (i, 0)),
    out_shape=jax.ShapeDtypeStruct((N_PAD, D), jnp.float32),
)


def _tc_b_body(sp_ref, g1_ref, degp_ref, b_ref, w_ref, g_ref):
    dinv = _dinv_from(degp_ref)
    agg = sp_ref[0] + sp_ref[1] + g1_ref[...]
    z = jnp.maximum(dinv * agg + b_ref[...], 0.0)
    g_ref[...] = dinv * jnp.dot(z, w_ref[...],
                                preferred_element_type=jnp.float32)


_tc_b = pl.pallas_call(
    _tc_b_body,
    grid=(N_PAD // BLK,),
    in_specs=[
        pl.BlockSpec((2, BLK, D), lambda i: (0, i, 0)),
        pl.BlockSpec((BLK, D), lambda i: (i, 0)),
        pl.BlockSpec((2, BLK, DEG_W), lambda i: (0, i, 0)),
        pl.BlockSpec((1, D), lambda i: (0, 0)),
        pl.BlockSpec((D, D), lambda i: (0, 0)),
    ],
    out_specs=pl.BlockSpec((BLK, D), lambda i: (i, 0)),
    out_shape=jax.ShapeDtypeStruct((N_PAD, D), jnp.float32),
)


def _tc_c_body(sp_ref, g2_ref, degp_ref, b_ref, o_ref):
    dinv = _dinv_from(degp_ref)
    agg = sp_ref[0] + sp_ref[1] + g2_ref[...]
    o_ref[...] = dinv * agg + b_ref[...]


_tc_c = pl.pallas_call(
    _tc_c_body,
    grid=(N_PAD // BLK,),
    in_specs=[
        pl.BlockSpec((2, BLK, D), lambda i: (0, i, 0)),
        pl.BlockSpec((BLK, D), lambda i: (i, 0)),
        pl.BlockSpec((2, BLK, DEG_W), lambda i: (0, i, 0)),
        pl.BlockSpec((1, D), lambda i: (0, 0)),
    ],
    out_specs=pl.BlockSpec((BLK, D), lambda i: (i, 0)),
    out_shape=jax.ShapeDtypeStruct((N_PAD, D), jnp.float32),
)


def kernel(x, edge_index, W1, b1, W2, b2):
    ei = edge_index.astype(jnp.int32)
    padv = jnp.asarray(
        DUMMY + np.arange(E_PAD - N_EDGES, dtype=np.int32) % (
            N_PAD - N_NODES))
    src = jnp.concatenate([ei[0], padv]).reshape(NW, CHUNKS_PER_W, CHUNK)
    dst = jnp.concatenate([ei[1], padv]).reshape(NW, CHUNKS_PER_W, CHUNK)

    x_pad = jnp.pad(x, ((0, N_PAD - N_NODES), (0, 0)))
    ones_deg = jnp.ones((CHUNK, DEG_W), jnp.float32)
    zeros_deg = jnp.zeros((ROWS_PER_TILE, DEG_W), jnp.float32)
    zeros_row = jnp.zeros((ROWS_PER_TILE, D), jnp.float32)

    degp = _deg_call(dst, ones_deg, zeros_deg).reshape(2, N_PAD, DEG_W)
    g1 = _tc_a(x_pad, W1, degp)
    s1 = _prop_call(g1, src, dst, zeros_row).reshape(2, N_PAD, D)
    g2 = _tc_b(s1, g1, degp, b1.reshape(1, D), W2)
    s2 = _prop_call(g2, src, dst, zeros_row).reshape(2, N_PAD, D)
    out = _tc_c(s2, g2, degp, b2.reshape(1, D))
    return out[:N_NODES]


# speedup vs baseline: 1.1596x; 1.0002x over previous
import functools

import numpy as np
import jax
import jax.numpy as jnp
from jax import lax
from jax.experimental import pallas as pl
from jax.experimental.pallas import tpu as pltpu
from jax.experimental.pallas import tpu_sc as plsc

N_NODES = 10000
D = 128
N_PAD = 10240
ROWS_PER_TILE = N_PAD // 16
N_EDGES = 320000
CHUNK = 128
NW = 32
CHUNKS_PER_W = 80
E_PAD = NW * CHUNKS_PER_W * CHUNK
DUMMY = N_NODES
DEG_W = 128

_mesh = plsc.VectorSubcoreMesh(core_axis_name="c", subcore_axis_name="s")


NBUF = 2
SBLK = 40
SBLOCKS = ((0, 40), (40, 40))


def _deg_body(dst_hbm, ones_hbm, zeros_hbm, out_hbm, dstv, onesv, sem, S):
    c = lax.axis_index("c")
    s = lax.axis_index("s")
    w = c * 16 + s
    r0 = s * ROWS_PER_TILE
    pltpu.sync_copy(zeros_hbm, S.at[pl.ds(r0, ROWS_PER_TILE)])
    pltpu.sync_copy(ones_hbm, onesv)
    pltpu.sync_copy(dst_hbm.at[w], dstv)
    plsc.subcore_barrier()

    def fire(i, carry):
        pltpu.async_copy(onesv, S.at[dstv.at[i]], sem, add=True)
        return carry

    lax.fori_loop(0, CHUNKS_PER_W, fire, 0)

    def drain(i, carry):
        pltpu.make_async_copy(onesv, S.at[dstv.at[i]], sem).wait()
        return carry

    lax.fori_loop(0, CHUNKS_PER_W, drain, 0)
    plsc.subcore_barrier()
    pltpu.sync_copy(S.at[pl.ds(r0, ROWS_PER_TILE)],
                    out_hbm.at[pl.ds(c * N_PAD + r0, ROWS_PER_TILE)])


_deg_call = functools.partial(
    pl.kernel,
    mesh=_mesh,
    out_type=jax.ShapeDtypeStruct((2 * N_PAD, DEG_W), jnp.float32),
    scratch_types=[
        pltpu.VMEM((CHUNKS_PER_W, CHUNK), jnp.int32),
        pltpu.VMEM((CHUNK, DEG_W), jnp.float32),
        pltpu.SemaphoreType.DMA,
        pltpu.VMEM_SHARED((N_PAD, DEG_W), jnp.float32),
    ],
)(_deg_body)


def _prop_body(g_hbm, src_hbm, dst_hbm, zeros_hbm, out_hbm,
               srcv, dstv, rows0, rows1, sem0, sem1, S):
    c = lax.axis_index("c")
    s = lax.axis_index("s")
    w = c * 16 + s
    r0 = s * ROWS_PER_TILE
    rows = (rows0, rows1)
    sems = (sem0, sem1)
    pltpu.sync_copy(zeros_hbm, S.at[pl.ds(r0, ROWS_PER_TILE)])
    pltpu.sync_copy(dst_hbm.at[w], dstv)
    plsc.subcore_barrier()

    for off, n in SBLOCKS:
        pltpu.sync_copy(src_hbm.at[w, pl.ds(off, n)],
                        srcv.at[pl.ds(0, n)])
        for b in range(NBUF):
            pltpu.async_copy(g_hbm.at[srcv.at[b]], rows[b], sems[b])

        def body(t, carry):
            for b in range(NBUF):
                lc = t * NBUF + b
                gc = off + lc
                pltpu.make_async_copy(g_hbm.at[srcv.at[lc]], rows[b],
                                      sems[b]).wait()
                pltpu.sync_copy(rows[b], S.at[dstv.at[gc]], add=True)
                nxt = lc + NBUF

                @pl.when(nxt < n)
                def _():
                    pltpu.async_copy(g_hbm.at[srcv.at[nxt]], rows[b],
                                     sems[b])
            return carry

        lax.fori_loop(0, n // NBUF, body, 0)

    plsc.subcore_barrier()
    pltpu.sync_copy(S.at[pl.ds(r0, ROWS_PER_TILE)],
                    out_hbm.at[pl.ds(c * N_PAD + r0, ROWS_PER_TILE)])


_prop_call = functools.partial(
    pl.kernel,
    mesh=_mesh,
    out_type=jax.ShapeDtypeStruct((2 * N_PAD, D), jnp.float32),
    scratch_types=[
        pltpu.VMEM((SBLK, CHUNK), jnp.int32),
        pltpu.VMEM((CHUNKS_PER_W, CHUNK), jnp.int32),
        pltpu.VMEM((CHUNK, D), jnp.float32),
        pltpu.VMEM((CHUNK, D), jnp.float32),
        pltpu.SemaphoreType.DMA,
        pltpu.SemaphoreType.DMA,
        pltpu.VMEM_SHARED((N_PAD, D), jnp.float32),
    ],
)(_prop_body)


BLK = 1280


def _dinv_from(degp_ref):
    deg = degp_ref[0, :, 0:1] + degp_ref[1, :, 0:1] + 1.0
    return lax.rsqrt(deg)


def _tc_a_body(x_ref, w_ref, degp_ref, g_ref):
    dinv = _dinv_from(degp_ref)
    g_ref[...] = dinv * jnp.dot(x_ref[...], w_ref[...],
                                preferred_element_type=jnp.float32)


_tc_a = pl.pallas_call(
    _tc_a_body,
    grid=(N_PAD // BLK,),
    in_specs=[
        pl.BlockSpec((BLK, D), lambda i: (i, 0)),
        pl.BlockSpec((D, D), lambda i: (0, 0)),
        pl.BlockSpec((2, BLK, DEG_W), lambda i: (0, i, 0)),
    ],
    out_specs=pl.BlockSpec((BLK, D), lambda i: (i, 0)),
    out_shape=jax.ShapeDtypeStruct((N_PAD, D), jnp.float32),
)


def _tc_b_body(sp_ref, g1_ref, degp_ref, b_ref, w_ref, g_ref):
    dinv = _dinv_from(degp_ref)
    agg = sp_ref[0] + sp_ref[1] + g1_ref[...]
    z = jnp.maximum(dinv * agg + b_ref[...], 0.0)
    g_ref[...] = dinv * jnp.dot(z, w_ref[...],
                                preferred_element_type=jnp.float32)


_tc_b = pl.pallas_call(
    _tc_b_body,
    grid=(N_PAD // BLK,),
    in_specs=[
        pl.BlockSpec((2, BLK, D), lambda i: (0, i, 0)),
        pl.BlockSpec((BLK, D), lambda i: (i, 0)),
        pl.BlockSpec((2, BLK, DEG_W), lambda i: (0, i, 0)),
        pl.BlockSpec((1, D), lambda i: (0, 0)),
        pl.BlockSpec((D, D), lambda i: (0, 0)),
    ],
    out_specs=pl.BlockSpec((BLK, D), lambda i: (i, 0)),
    out_shape=jax.ShapeDtypeStruct((N_PAD, D), jnp.float32),
)


def _tc_c_body(sp_ref, g2_ref, degp_ref, b_ref, o_ref):
    dinv = _dinv_from(degp_ref)
    agg = sp_ref[0] + sp_ref[1] + g2_ref[...]
    o_ref[...] = dinv * agg + b_ref[...]


_tc_c = pl.pallas_call(
    _tc_c_body,
    grid=(N_PAD // BLK,),
    in_specs=[
        pl.BlockSpec((2, BLK, D), lambda i: (0, i, 0)),
        pl.BlockSpec((BLK, D), lambda i: (i, 0)),
        pl.BlockSpec((2, BLK, DEG_W), lambda i: (0, i, 0)),
        pl.BlockSpec((1, D), lambda i: (0, 0)),
    ],
    out_specs=pl.BlockSpec((BLK, D), lambda i: (i, 0)),
    out_shape=jax.ShapeDtypeStruct((N_PAD, D), jnp.float32),
)


def kernel(x, edge_index, W1, b1, W2, b2):
    ei = edge_index.astype(jnp.int32)
    padv = jnp.asarray(
        DUMMY + np.arange(E_PAD - N_EDGES, dtype=np.int32) % (
            N_PAD - N_NODES))
    src = jnp.concatenate([ei[0], padv]).reshape(NW, CHUNKS_PER_W, CHUNK)
    dst = jnp.concatenate([ei[1], padv]).reshape(NW, CHUNKS_PER_W, CHUNK)

    x_pad = jnp.pad(x, ((0, N_PAD - N_NODES), (0, 0)))
    ones_deg = jnp.ones((CHUNK, DEG_W), jnp.float32)
    zeros_deg = jnp.zeros((ROWS_PER_TILE, DEG_W), jnp.float32)
    zeros_row = jnp.zeros((ROWS_PER_TILE, D), jnp.float32)

    degp = _deg_call(dst, ones_deg, zeros_deg).reshape(2, N_PAD, DEG_W)
    g1 = _tc_a(x_pad, W1, degp)
    s1 = _prop_call(g1, src, dst, zeros_row).reshape(2, N_PAD, D)
    g2 = _tc_b(s1, g1, degp, b1.reshape(1, D), W2)
    s2 = _prop_call(g2, src, dst, zeros_row).reshape(2, N_PAD, D)
    out = _tc_c(s2, g2, degp, b2.reshape(1, D))
    return out[:N_NODES]
